# baseline (speedup 1.0000x reference)
import functools

import jax
import jax.numpy as jnp
from jax import lax
from jax.experimental import pallas as pl
from jax.experimental.pallas import tpu as pltpu
from jax.experimental.pallas import tpu_sc as plsc

HIST_ROWS = 10
NUM_WORKERS = 32
LANES = 16
IDX_CHUNK = 128

SC_ROWS = 1024
TC_BLOCK = 1024
CP_BLOCK = 512


def _build_sc_kernel(n_rows, d, row0, sc_rows):
    rows_per_w = sc_rows // NUM_WORKERS
    chunk = min(IDX_CHUNK, rows_per_w)
    n_chunks = rows_per_w // chunk
    mesh = plsc.VectorSubcoreMesh(core_axis_name="c", subcore_axis_name="s",
                                  num_cores=2)

    @functools.partial(
        pl.kernel,
        mesh=mesh,
        out_type=jax.ShapeDtypeStruct((sc_rows, d), jnp.float32),
        scratch_types=[
            pltpu.VMEM((rows_per_w, d), jnp.float32),
            pltpu.VMEM((HIST_ROWS, n_chunks, chunk), jnp.int32),
            pltpu.SemaphoreType.DMA,
        ],
    )
    def sc_sum(ap_hbm, hist_hbm, out_hbm, acc, idx, sem):
        wid = lax.axis_index("s") * 2 + lax.axis_index("c")
        base = wid * rows_per_w

        iota = lax.iota(jnp.int32, LANES)
        for r in range(HIST_ROWS):
            for j in range(n_chunks):
                for l in range(chunk // LANES):
                    off = r * n_rows + row0 + j * chunk + l * LANES
                    idx[r, j, pl.ds(l * LANES, LANES)] = iota + (base + off)

        pltpu.sync_copy(ap_hbm.at[pl.ds(row0 + base, rows_per_w)], acc)

        copies = []
        for r in range(HIST_ROWS):
            for j in range(n_chunks):
                copies.append(
                    pltpu.async_copy(
                        hist_hbm.at[idx.at[r, j]],
                        acc.at[pl.ds(j * chunk, chunk)],
                        sem,
                        add=True,
                    )
                )
        for cp in copies:
            cp.wait()

        pltpu.sync_copy(acc, out_hbm.at[pl.ds(base, rows_per_w)])

    return sc_sum


def _tc_body(ap_ref, hist_ref, out_ref):
    out_ref[...] = ap_ref[...] + jnp.sum(hist_ref[...], axis=0)


def _tc_sum(ap, hist, n_rows, d, tc_rows):
    grid = (tc_rows // TC_BLOCK,)
    return pl.pallas_call(
        _tc_body,
        grid=grid,
        in_specs=[
            pl.BlockSpec((TC_BLOCK, d), lambda i: (i, 0)),
            pl.BlockSpec((HIST_ROWS, TC_BLOCK, d), lambda i: (0, i, 0)),
        ],
        out_specs=pl.BlockSpec((TC_BLOCK, d), lambda i: (i, 0)),
        out_shape=jax.ShapeDtypeStruct((n_rows, d), jnp.float32),
    )(ap, hist)


def _merge_body(tc_ref, sc_ref, out_ref):
    out_ref[...] = sc_ref[...]


def _merge(tc_full, sc_out, n_rows, d, row0):
    grid = ((n_rows - row0) // CP_BLOCK,)
    return pl.pallas_call(
        _merge_body,
        grid=grid,
        in_specs=[
            pl.BlockSpec((8, d), lambda i: (0, 0)),
            pl.BlockSpec((CP_BLOCK, d), lambda i: (i, 0)),
        ],
        out_specs=pl.BlockSpec((CP_BLOCK, d), lambda i: (row0 // CP_BLOCK + i, 0)),
        out_shape=jax.ShapeDtypeStruct((n_rows, d), jnp.float32),
        input_output_aliases={0: 0},
    )(tc_full, sc_out)


@jax.jit
def kernel(action_potential, action_potential_history):
    n_rows, d = action_potential.shape
    tc_rows = n_rows - SC_ROWS
    hist2d = action_potential_history.reshape(-1, d)
    sc_out = _build_sc_kernel(n_rows, d, tc_rows, SC_ROWS)(
        action_potential, hist2d
    )
    tc_full = _tc_sum(action_potential, action_potential_history, n_rows, d,
                      tc_rows)
    return _merge(tc_full, sc_out, n_rows, d, tc_rows)


# speedup vs baseline: 1.0320x; 1.0320x over previous
import functools

import jax
import jax.numpy as jnp
from jax import lax
from jax.experimental import pallas as pl
from jax.experimental.pallas import tpu as pltpu
from jax.experimental.pallas import tpu_sc as plsc

HIST_ROWS = 10
NUM_WORKERS = 16
LANES = 16
IDX_CHUNK = 128

SC_ROWS = 1024
TC_BLOCK = 1024
CP_BLOCK = 512


def _build_sc_kernel(n_rows, d, row0, sc_rows):
    rows_per_w = sc_rows // NUM_WORKERS
    chunk = min(IDX_CHUNK, rows_per_w)
    n_chunks = rows_per_w // chunk
    mesh = plsc.VectorSubcoreMesh(core_axis_name="c", subcore_axis_name="s",
                                  num_cores=1)

    @functools.partial(
        pl.kernel,
        mesh=mesh,
        out_type=jax.ShapeDtypeStruct((sc_rows, d), jnp.float32),
        scratch_types=[
            pltpu.VMEM((rows_per_w, d), jnp.float32),
            pltpu.VMEM((HIST_ROWS, n_chunks, chunk), jnp.int32),
            pltpu.SemaphoreType.DMA,
        ],
    )
    def sc_sum(ap_hbm, hist_hbm, out_hbm, acc, idx, sem):
        wid = lax.axis_index("s") + lax.axis_index("c")
        base = wid * rows_per_w

        iota = lax.iota(jnp.int32, LANES)
        for r in range(HIST_ROWS):
            for j in range(n_chunks):
                for l in range(chunk // LANES):
                    off = r * n_rows + row0 + j * chunk + l * LANES
                    idx[r, j, pl.ds(l * LANES, LANES)] = iota + (base + off)

        pltpu.sync_copy(ap_hbm.at[pl.ds(row0 + base, rows_per_w)], acc)

        copies = []
        for r in range(HIST_ROWS):
            for j in range(n_chunks):
                copies.append(
                    pltpu.async_copy(
                        hist_hbm.at[idx.at[r, j]],
                        acc.at[pl.ds(j * chunk, chunk)],
                        sem,
                        add=True,
                    )
                )
        for cp in copies:
            cp.wait()

        pltpu.sync_copy(acc, out_hbm.at[pl.ds(base, rows_per_w)])

    return sc_sum


def _tc_body(ap_ref, hist_ref, out_ref):
    out_ref[...] = ap_ref[...] + jnp.sum(hist_ref[...], axis=0)


def _tc_sum(ap, hist, n_rows, d, tc_rows):
    grid = (tc_rows // TC_BLOCK,)
    return pl.pallas_call(
        _tc_body,
        grid=grid,
        in_specs=[
            pl.BlockSpec((TC_BLOCK, d), lambda i: (i, 0)),
            pl.BlockSpec((HIST_ROWS, TC_BLOCK, d), lambda i: (0, i, 0)),
        ],
        out_specs=pl.BlockSpec((TC_BLOCK, d), lambda i: (i, 0)),
        out_shape=jax.ShapeDtypeStruct((n_rows, d), jnp.float32),
    )(ap, hist)


def _merge_body(tc_ref, sc_ref, out_ref):
    out_ref[...] = sc_ref[...]


def _merge(tc_full, sc_out, n_rows, d, row0):
    grid = ((n_rows - row0) // CP_BLOCK,)
    return pl.pallas_call(
        _merge_body,
        grid=grid,
        in_specs=[
            pl.BlockSpec((8, d), lambda i: (0, 0)),
            pl.BlockSpec((CP_BLOCK, d), lambda i: (i, 0)),
        ],
        out_specs=pl.BlockSpec((CP_BLOCK, d), lambda i: (row0 // CP_BLOCK + i, 0)),
        out_shape=jax.ShapeDtypeStruct((n_rows, d), jnp.float32),
        input_output_aliases={0: 0},
    )(tc_full, sc_out)


@jax.jit
def kernel(action_potential, action_potential_history):
    n_rows, d = action_potential.shape
    tc_rows = n_rows - SC_ROWS
    hist2d = action_potential_history.reshape(-1, d)
    sc_out = _build_sc_kernel(n_rows, d, tc_rows, SC_ROWS)(
        action_potential, hist2d
    )
    tc_full = _tc_sum(action_potential, action_potential_history, n_rows, d,
                      tc_rows)
    return _merge(tc_full, sc_out, n_rows, d, tc_rows)
